# Initial kernel scaffold; baseline (speedup 1.0000x reference)
import jax
import jax.numpy as jnp
from jax.experimental import pallas as pl


def kernel(input_ids, token_type_ids, word_emb, pos_emb, type_emb, ln_w, ln_b):
    raise NotImplementedError("write your pallas kernel here")


# speedup vs baseline: 1.4972x; 1.4972x over previous
import functools

import jax
import jax.numpy as jnp
from jax import lax
from jax.experimental import pallas as pl
from jax.experimental.pallas import tpu as pltpu
from jax.experimental.pallas import tpu_sc as plsc

EPS = 1e-12
NUM_SC_CORES = 2
NUM_SC_SUBCORES = 16
NW = NUM_SC_CORES * NUM_SC_SUBCORES
GATHER_CHUNK = 64
TOK_BLOCK = 256


def _sc_gather(word_emb, flat_ids):
    n_tok = flat_ids.shape[0]
    hidden = word_emb.shape[1]
    per_w = n_tok // NW
    n_chunks = per_w // GATHER_CHUNK
    mesh = plsc.VectorSubcoreMesh(core_axis_name="c", subcore_axis_name="s")

    @functools.partial(
        pl.kernel,
        out_type=jax.ShapeDtypeStruct((n_tok, hidden), jnp.float32),
        mesh=mesh,
        scratch_types=[
            pltpu.VMEM((GATHER_CHUNK,), jnp.int32),
            pltpu.VMEM((GATHER_CHUNK, hidden), jnp.float32),
            pltpu.SemaphoreType.DMA,
        ],
    )
    def gather_kernel(table_hbm, idx_hbm, out_hbm, idx_v, rows_v, sem):
        wid = lax.axis_index("s") * NUM_SC_CORES + lax.axis_index("c")
        base = wid * per_w
        for c in range(n_chunks):
            off = base + c * GATHER_CHUNK
            pltpu.sync_copy(idx_hbm.at[pl.ds(off, GATHER_CHUNK)], idx_v)
            pltpu.async_copy(table_hbm.at[idx_v], rows_v, sem).wait()
            pltpu.sync_copy(rows_v, out_hbm.at[pl.ds(off, GATHER_CHUNK)])

    return gather_kernel(word_emb, flat_ids)


def _tc_add_ln(gathered, pos_emb, type_emb, tt_col, ln_w2, ln_b2):
    n_tok, hidden = gathered.shape
    seq = pos_emb.shape[0]
    blocks_per_seq = seq // TOK_BLOCK

    def body(g_ref, pos_ref, type_ref, tt_ref, w_ref, b_ref, o_ref):
        i = pl.program_id(0)
        s_off = lax.rem(i, blocks_per_seq) * TOK_BLOCK
        t0 = type_ref[0:1, :]
        dt = type_ref[1:2, :] - t0
        e = (
            g_ref[...]
            + pos_ref[pl.ds(s_off, TOK_BLOCK), :]
            + t0
            + tt_ref[...] * dt
        )
        mean = jnp.mean(e, axis=1, keepdims=True)
        ec = e - mean
        var = jnp.mean(ec * ec, axis=1, keepdims=True)
        o_ref[...] = ec * lax.rsqrt(var + EPS) * w_ref[...] + b_ref[...]

    return pl.pallas_call(
        body,
        grid=(n_tok // TOK_BLOCK,),
        in_specs=[
            pl.BlockSpec((TOK_BLOCK, hidden), lambda i: (i, 0)),
            pl.BlockSpec((seq, hidden), lambda i: (0, 0)),
            pl.BlockSpec((2, hidden), lambda i: (0, 0)),
            pl.BlockSpec((TOK_BLOCK, 1), lambda i: (i, 0)),
            pl.BlockSpec((1, hidden), lambda i: (0, 0)),
            pl.BlockSpec((1, hidden), lambda i: (0, 0)),
        ],
        out_specs=pl.BlockSpec((TOK_BLOCK, hidden), lambda i: (i, 0)),
        out_shape=jax.ShapeDtypeStruct((n_tok, hidden), jnp.float32),
    )(gathered, pos_emb, type_emb, tt_col, ln_w2, ln_b2)


def kernel(input_ids, token_type_ids, word_emb, pos_emb, type_emb, ln_w, ln_b):
    b, s = input_ids.shape
    hidden = word_emb.shape[1]
    flat_ids = input_ids.reshape(-1)
    gathered = _sc_gather(word_emb, flat_ids)
    tt_col = token_type_ids.reshape(-1, 1).astype(jnp.float32)
    out = _tc_add_ln(
        gathered,
        pos_emb[:s],
        type_emb,
        tt_col,
        ln_w.reshape(1, hidden),
        ln_b.reshape(1, hidden),
    )
    return out.reshape(b, s, hidden)
